# Initial kernel scaffold; baseline (speedup 1.0000x reference)
import jax
import jax.numpy as jnp
from jax.experimental import pallas as pl


def kernel(x, edge_index, edge_type, W0, b0, loop0, W1, b1, loop1):
    raise NotImplementedError("write your pallas kernel here")


# speedup vs baseline: 16.5996x; 16.5996x over previous
import functools

import jax
import jax.numpy as jnp
from jax import lax
from jax.experimental import pallas as pl
from jax.experimental.pallas import tpu as pltpu
from jax.experimental.pallas import tpu_sc as plsc

NC = 2
NS = 16
NW = NC * NS
BLK = 128
ROWBLK = 128
TC_ROWS = 1000


def _sc_gather_scatter(table, gidx, didx, npad, h):
  k = gidx.shape[1]
  rows_per_sub = npad // NS
  chunks = rows_per_sub // ROWBLK

  def body(table_ref, gidx_ref, didx_ref, parts_ref,
           agg, gidx_v, didx_v, rowbuf, sem):
    c = lax.axis_index("c")
    s = lax.axis_index("s")
    w = s * NC + c

    def zb(q, carry):
      rowbuf[q // 8, pl.ds((q % 8) * 16, 16)] = jnp.zeros((16,), jnp.float32)
      return carry
    lax.fori_loop(0, ROWBLK * 8, zb, 0)
    for t in range(chunks):
      pltpu.sync_copy(rowbuf, agg.at[pl.ds(s * rows_per_sub + t * ROWBLK,
                                           ROWBLK)])

    pltpu.sync_copy(gidx_ref.at[w], gidx_v)
    pltpu.sync_copy(didx_ref.at[w], didx_v)
    plsc.subcore_barrier()

    def blk(j, carry):
      pltpu.async_copy(table_ref.at[gidx_v.at[j]], rowbuf, sem).wait()
      pltpu.sync_copy(rowbuf, agg.at[didx_v.at[j]], add=True)
      return carry
    lax.fori_loop(0, k, blk, 0)
    plsc.subcore_barrier()

    for t in range(chunks):
      r0 = s * rows_per_sub + t * ROWBLK
      pltpu.sync_copy(agg.at[pl.ds(r0, ROWBLK)], rowbuf)
      pltpu.sync_copy(rowbuf, parts_ref.at[c, pl.ds(r0, ROWBLK)])

  mesh = plsc.VectorSubcoreMesh(core_axis_name="c", subcore_axis_name="s")
  return pl.kernel(
      body,
      out_type=jax.ShapeDtypeStruct((NC, npad, h), jnp.float32),
      mesh=mesh,
      scratch_types=[
          pltpu.VMEM_SHARED((npad, h), jnp.float32),
          pltpu.VMEM((k, BLK), jnp.int32),
          pltpu.VMEM((k, BLK), jnp.int32),
          pltpu.VMEM((BLK, h), jnp.float32),
          pltpu.SemaphoreType.DMA,
      ],
  )(table, gidx, didx)


def _proj_body(r, x_ref, w_ref, loop_ref, b_ref, xp_ref, sl_ref):
  xb = x_ref[...]
  for i in range(r):
    xp_ref[i] = jnp.dot(xb, w_ref[i], preferred_element_type=jnp.float32)
  sl_ref[...] = (jnp.dot(xb, loop_ref[...], preferred_element_type=jnp.float32)
                 + b_ref[...])


def _mid_body(r, p_ref, sl_ref, w_ref, loop_ref, b_ref, xp_ref, sl1_ref):
  hb = jnp.maximum(p_ref[0] + p_ref[1] + sl_ref[...], 0.0)
  for i in range(r):
    xp_ref[i] = jnp.dot(hb, w_ref[i], preferred_element_type=jnp.float32)
  sl1_ref[...] = (jnp.dot(hb, loop_ref[...],
                          preferred_element_type=jnp.float32) + b_ref[...])


def _fin_body(p_ref, sl_ref, out_ref):
  out_ref[...] = p_ref[0] + p_ref[1] + sl_ref[...]


def kernel(x, edge_index, edge_type, W0, b0, loop0, W1, b1, loop1):
  n, d = x.shape
  e = edge_type.shape[0]
  r, _, h = W0.shape
  assert n % TC_ROWS == 0
  grid = n // TC_ROWS

  epad = -(-e // (NW * BLK)) * (NW * BLK)
  k = epad // (NW * BLK)
  npad = -(-(n + 1) // (NS * ROWBLK)) * (NS * ROWBLK)

  src = edge_index[0].astype(jnp.int32)
  dst = edge_index[1].astype(jnp.int32)
  et = edge_type.astype(jnp.int32)
  pad = epad - e
  src = jnp.concatenate([src, jnp.zeros((pad,), jnp.int32)])
  et = jnp.concatenate([et, jnp.zeros((pad,), jnp.int32)])
  dst = jnp.concatenate([dst, jnp.full((pad,), n, jnp.int32)])
  gidx = (et * n + src).reshape(NW, k, BLK)
  didx = dst.reshape(NW, k, BLK)

  wfull = pl.BlockSpec((r, d, h), lambda i: (0, 0, 0))
  lfull = pl.BlockSpec((d, h), lambda i: (0, 0))
  bfull = pl.BlockSpec((1, h), lambda i: (0, 0))
  rowblk = pl.BlockSpec((TC_ROWS, d), lambda i: (i, 0))
  xpblk = pl.BlockSpec((r, TC_ROWS, h), lambda i: (0, i, 0))

  proj = pl.pallas_call(
      functools.partial(_proj_body, r),
      grid=(grid,),
      in_specs=[rowblk, wfull, lfull, bfull],
      out_specs=[xpblk, rowblk],
      out_shape=[jax.ShapeDtypeStruct((r, n, h), jnp.float32),
                 jax.ShapeDtypeStruct((n, h), jnp.float32)],
  )
  mid = pl.pallas_call(
      functools.partial(_mid_body, r),
      grid=(grid,),
      in_specs=[pl.BlockSpec((NC, TC_ROWS, h), lambda i: (0, i, 0)),
                rowblk, wfull, lfull, bfull],
      out_specs=[xpblk, rowblk],
      out_shape=[jax.ShapeDtypeStruct((r, n, h), jnp.float32),
                 jax.ShapeDtypeStruct((n, h), jnp.float32)],
  )
  fin = pl.pallas_call(
      _fin_body,
      grid=(grid,),
      in_specs=[pl.BlockSpec((NC, TC_ROWS, h), lambda i: (0, i, 0)), rowblk],
      out_specs=rowblk,
      out_shape=jax.ShapeDtypeStruct((n, h), jnp.float32),
  )

  b0r = b0.reshape(1, h)
  b1r = b1.reshape(1, h)

  xp0, sl0 = proj(x, W0, loop0, b0r)
  parts0 = _sc_gather_scatter(xp0.reshape(r * n, h), gidx, didx, npad, h)
  xp1, sl1 = mid(parts0, sl0, W1, loop1, b1r)
  parts1 = _sc_gather_scatter(xp1.reshape(r * n, h), gidx, didx, npad, h)
  return fin(parts1, sl1)
